# Initial kernel scaffold; baseline (speedup 1.0000x reference)
import jax
import jax.numpy as jnp
from jax.experimental import pallas as pl


def kernel(x, edge_index, W0, b0, W1, b1, W2, b2):
    raise NotImplementedError("write your pallas kernel here")


# speedup vs baseline: 4.6421x; 4.6421x over previous
import functools

import jax
import jax.numpy as jnp
from jax import lax
from jax.experimental import pallas as pl
from jax.experimental.pallas import tpu as pltpu
from jax.experimental.pallas import tpu_sc as plsc

N_NODES = 10000
D = 128
N_CLASSES = 40
E = 320000

NC = 2
NS = 16
NW = NC * NS

CHUNK = 128
CHUNKS_PER_TILE = 79
E_PAD = NW * CHUNKS_PER_TILE * CHUNK
N_PAD = 10240
ROWS_PER_TILE = N_PAD // NS

_mesh = plsc.VectorSubcoreMesh(core_axis_name="c", subcore_axis_name="s")


DEG_CHUNKS = E_PAD // (NS * CHUNK)


@functools.partial(
    pl.kernel,
    mesh=_mesh,
    out_type=jax.ShapeDtypeStruct((NC, N_PAD, D), jnp.float32),
    scratch_types=[
        pltpu.VMEM((DEG_CHUNKS, CHUNK), jnp.int32),
        pltpu.VMEM((CHUNK, D), jnp.float32),
        pltpu.VMEM_SHARED((N_PAD, D), jnp.float32),
    ],
)
def _sc_degrees(sd_hbm, out_hbm, idx_v, buf_v, deg_sh):
    c = lax.axis_index("c")
    s = lax.axis_index("s")
    pltpu.sync_copy(sd_hbm.at[c * NS + s], idx_v)

    def zr(i, _):
        buf_v[i // 8, pl.ds((i % 8) * 16, 16)] = jnp.zeros((16,), jnp.float32)
        return 0
    lax.fori_loop(0, CHUNK * D // 16, zr, 0)
    for k in range(ROWS_PER_TILE // CHUNK):
        pltpu.sync_copy(buf_v,
                        deg_sh.at[pl.ds(s * ROWS_PER_TILE + k * CHUNK, CHUNK)])

    def fill(i, _):
        buf_v[i // 8, pl.ds((i % 8) * 16, 16)] = jnp.full((16,), 1.0, jnp.float32)
        return 0
    lax.fori_loop(0, CHUNK * D // 16, fill, 0)
    plsc.subcore_barrier()

    def body(j, _):
        pltpu.sync_copy(buf_v, deg_sh.at[idx_v.at[j]], add=True)
        return 0
    lax.fori_loop(0, DEG_CHUNKS, body, 0)
    plsc.subcore_barrier()

    pltpu.sync_copy(deg_sh.at[pl.ds(s * ROWS_PER_TILE, ROWS_PER_TILE)],
                    out_hbm.at[c].at[pl.ds(s * ROWS_PER_TILE, ROWS_PER_TILE)])


@functools.partial(
    pl.kernel,
    mesh=_mesh,
    out_type=jax.ShapeDtypeStruct((NC, N_PAD, D), jnp.float32),
    scratch_types=[
        pltpu.VMEM((CHUNKS_PER_TILE, CHUNK), jnp.int32),
        pltpu.VMEM((CHUNKS_PER_TILE, CHUNK), jnp.int32),
        pltpu.VMEM((CHUNK, D), jnp.float32),
        pltpu.VMEM_SHARED((N_PAD, D), jnp.float32),
        pltpu.SemaphoreType.DMA,
    ],
)
def _sc_edge_agg(table_hbm, srcs_hbm, dsts_hbm, out_hbm, src_v, dst_v, rows_v,
                 agg_sh, sem):
    c = lax.axis_index("c")
    s = lax.axis_index("s")
    wid = c * NS + s
    pltpu.sync_copy(srcs_hbm.at[wid], src_v)
    pltpu.sync_copy(dsts_hbm.at[wid], dst_v)

    def zr(i, _):
        rows_v[i // 8, pl.ds((i % 8) * 16, 16)] = jnp.zeros((16,), jnp.float32)
        return 0
    lax.fori_loop(0, CHUNK * D // 16, zr, 0)
    for k in range(ROWS_PER_TILE // CHUNK):
        pltpu.sync_copy(rows_v,
                        agg_sh.at[pl.ds(s * ROWS_PER_TILE + k * CHUNK, CHUNK)])
    plsc.subcore_barrier()

    def body(j, _):
        pltpu.async_copy(table_hbm.at[src_v.at[j]], rows_v, sem).wait()
        pltpu.sync_copy(rows_v, agg_sh.at[dst_v.at[j]], add=True)
        return 0
    lax.fori_loop(0, CHUNKS_PER_TILE, body, 0)
    plsc.subcore_barrier()

    pltpu.sync_copy(agg_sh.at[pl.ds(s * ROWS_PER_TILE, ROWS_PER_TILE)],
                    out_hbm.at[c].at[pl.ds(s * ROWS_PER_TILE, ROWS_PER_TILE)])


def _tc_prologue_body(x_ref, degs_ref, w_ref, t_ref, onorm_ref, inorm_ref):
    od = degs_ref[0, :, 0:1]
    idg = degs_ref[1, :, 0:1]
    onorm = jnp.where(od > 0, lax.rsqrt(od), 0.0)
    inorm = jnp.where(idg > 0, lax.rsqrt(idg), 0.0)
    onorm_ref[...] = onorm
    inorm_ref[...] = inorm
    t_ref[...] = jnp.dot(x_ref[...] * onorm, w_ref[...],
                         preferred_element_type=jnp.float32)


def _layernorm_relu(p_ref, inorm_ref, b_ref):
    h = (p_ref[0] + p_ref[1]) * inorm_ref[...] + b_ref[...]
    h = jnp.maximum(h, 0.0)
    rows = lax.broadcasted_iota(jnp.int32, (N_PAD, D), 0)
    mask = rows < N_NODES
    cnt = float(N_NODES * D)
    mu = jnp.sum(jnp.where(mask, h, 0.0)) / cnt
    var = jnp.sum(jnp.where(mask, (h - mu) ** 2, 0.0)) / cnt
    return (h - mu) * lax.rsqrt(var + 1e-5)


def _tc_mid_body(p_ref, inorm_ref, onorm_ref, b_ref, w_ref, t_ref):
    h = _layernorm_relu(p_ref, inorm_ref, b_ref)
    t_ref[...] = jnp.dot(h * onorm_ref[...], w_ref[...],
                         preferred_element_type=jnp.float32)


def _tc_mid_nomm_body(p_ref, inorm_ref, onorm_ref, b_ref, t_ref):
    h = _layernorm_relu(p_ref, inorm_ref, b_ref)
    t_ref[...] = h * onorm_ref[...]


def _tc_epilogue_body(p_ref, inorm_ref, w_ref, b_ref, out_ref):
    agg = ((p_ref[0] + p_ref[1]) * inorm_ref[...])[:N_NODES, :]
    out_ref[...] = jnp.dot(agg, w_ref[...],
                           preferred_element_type=jnp.float32) + b_ref[...]


_tc_prologue = pl.pallas_call(
    _tc_prologue_body,
    out_shape=(jax.ShapeDtypeStruct((N_PAD, D), jnp.float32),
               jax.ShapeDtypeStruct((N_PAD, 1), jnp.float32),
               jax.ShapeDtypeStruct((N_PAD, 1), jnp.float32)),
)

_tc_mid = pl.pallas_call(
    _tc_mid_body,
    out_shape=jax.ShapeDtypeStruct((N_PAD, D), jnp.float32),
)

_tc_mid_nomm = pl.pallas_call(
    _tc_mid_nomm_body,
    out_shape=jax.ShapeDtypeStruct((N_PAD, D), jnp.float32),
)

_tc_epilogue = pl.pallas_call(
    _tc_epilogue_body,
    out_shape=jax.ShapeDtypeStruct((N_NODES, N_CLASSES), jnp.float32),
)


def kernel(x, edge_index, W0, b0, W1, b1, W2, b2):
    src = edge_index[0].astype(jnp.int32)
    dst = edge_index[1].astype(jnp.int32)
    pad = jnp.full((E_PAD - E,), N_NODES, jnp.int32)
    srcs = jnp.concatenate([src, pad]).reshape(NW, CHUNKS_PER_TILE, CHUNK)
    dsts = jnp.concatenate([dst, pad]).reshape(NW, CHUNKS_PER_TILE, CHUNK)
    x_pad = jnp.concatenate(
        [x, jnp.zeros((N_PAD - N_NODES, D), jnp.float32)], axis=0)

    sd = jnp.concatenate([src, pad, dst, pad]).reshape(NW, DEG_CHUNKS, CHUNK)
    degs = _sc_degrees(sd)

    t0, onorm, inorm = _tc_prologue(x_pad, degs, W0)
    p0 = _sc_edge_agg(t0, srcs, dsts)
    t1 = _tc_mid(p0, inorm, onorm, b0.reshape(1, D), W1)
    p1 = _sc_edge_agg(t1, srcs, dsts)
    t2 = _tc_mid_nomm(p1, inorm, onorm, b1.reshape(1, D))
    p2 = _sc_edge_agg(t2, srcs, dsts)
    return _tc_epilogue(p2, inorm, W2, b2.reshape(1, N_CLASSES))
